# Initial kernel scaffold; baseline (speedup 1.0000x reference)
import jax
import jax.numpy as jnp
from jax.experimental import pallas as pl


def kernel(hidden_states, weight):
    raise NotImplementedError("write your pallas kernel here")


# speedup vs baseline: 1.0627x; 1.0627x over previous
import functools

import jax
import jax.numpy as jnp
from jax.experimental import pallas as pl
from jax.experimental.pallas import tpu as pltpu

_E = 16
_K = 2
_ALPHA = 0.01


def _gate_kernel(x_ref, w_ref, idx_ref, wgt_ref, aux_ref, acc_s, acc_c, *,
                 n_tokens):
    i = pl.program_id(0)
    nb = pl.num_programs(0)

    x = x_ref[...]
    w = w_ref[...]
    logits = jax.lax.dot_general(
        x, w, (((1,), (1,)), ((), ())),
        preferred_element_type=jnp.float32)

    m1 = jnp.max(logits, axis=-1, keepdims=True)
    ex = jnp.exp(logits - m1)
    denom = jnp.sum(ex, axis=-1, keepdims=True)
    scores = ex / denom

    iota = jax.lax.broadcasted_iota(jnp.int32, logits.shape, 1)
    i1 = jnp.min(jnp.where(logits == m1, iota, _E), axis=-1, keepdims=True)
    top1_mask = iota == i1
    l2 = jnp.where(top1_mask, -jnp.inf, logits)
    m2 = jnp.max(l2, axis=-1, keepdims=True)
    i2 = jnp.min(jnp.where(l2 == m2, iota, _E), axis=-1, keepdims=True)

    s1 = jnp.max(scores, axis=-1, keepdims=True)
    s2 = jnp.max(jnp.where(top1_mask, -1.0, scores), axis=-1, keepdims=True)

    idx_ref[...] = jnp.concatenate([i1, i2], axis=-1)
    wgt_ref[...] = jnp.concatenate([s1, s2], axis=-1)

    @pl.when(i == 0)
    def _():
        acc_s[...] = jnp.zeros_like(acc_s)
        acc_c[...] = jnp.zeros_like(acc_c)

    acc_s[...] += jnp.sum(scores, axis=0, keepdims=True)
    cnt = (top1_mask.astype(jnp.float32)
           + (iota == i2).astype(jnp.float32))
    acc_c[...] += jnp.sum(cnt, axis=0, keepdims=True)

    @pl.when(i == nb - 1)
    def _():
        scale = _ALPHA * _E / (float(n_tokens) * float(n_tokens) * _K)
        aux_ref[0, 0] = jnp.sum(acc_s[...] * acc_c[...]) * scale


def kernel(hidden_states, weight):
    bsz, seq_len, h = hidden_states.shape
    n = bsz * seq_len
    x = hidden_states.reshape(n, h)

    block = 2048
    nb = n // block

    idx, wgt, aux = pl.pallas_call(
        functools.partial(_gate_kernel, n_tokens=n),
        grid=(nb,),
        in_specs=[
            pl.BlockSpec((block, h), lambda i: (i, 0)),
            pl.BlockSpec((_E, h), lambda i: (0, 0)),
        ],
        out_specs=[
            pl.BlockSpec((block, _K), lambda i: (i, 0)),
            pl.BlockSpec((block, _K), lambda i: (i, 0)),
            pl.BlockSpec(memory_space=pltpu.SMEM),
        ],
        out_shape=[
            jax.ShapeDtypeStruct((n, _K), jnp.int32),
            jax.ShapeDtypeStruct((n, _K), jnp.float32),
            jax.ShapeDtypeStruct((1, 1), jnp.float32),
        ],
        scratch_shapes=[
            pltpu.VMEM((1, _E), jnp.float32),
            pltpu.VMEM((1, _E), jnp.float32),
        ],
    )(x, weight)

    return idx, wgt, aux[0, 0]
